# Initial kernel scaffold; baseline (speedup 1.0000x reference)
import jax
import jax.numpy as jnp
from jax.experimental import pallas as pl


def kernel(input_, target):
    raise NotImplementedError("write your pallas kernel here")


# speedup vs baseline: 4.3593x; 4.3593x over previous
import functools

import jax
import jax.numpy as jnp
from jax import lax
from jax.experimental import pallas as pl
from jax.experimental.pallas import tpu as pltpu
from jax.experimental.pallas import tpu_sc as plsc

DELTA_VAR = 0.5
DELTA_DIST = 2.0
GAMMA = 0.001
EPS = 1e-12
C = 64
L = 16
NC, NS = 2, 16
NW = NC * NS
CHUNK = 1024

_INTERPRET = False


def _sc_mesh():
    return plsc.VectorSubcoreMesh(
        core_axis_name="c", subcore_axis_name="s", num_cores=NC, num_subcores=NS
    )


def _seg_stats(x, t, E, P):
    NPX = P // NW
    NCHUNK = NPX // CHUNK
    G = CHUNK // L
    CE = C * E

    @functools.partial(
        pl.kernel,
        out_type=[
            jax.ShapeDtypeStruct((NW, CE), jnp.float32),
            jax.ShapeDtypeStruct((NW, C), jnp.float32),
        ],
        mesh=_sc_mesh(),
        scratch_types=[
            pltpu.VMEM((E, CHUNK), jnp.float32),
            pltpu.VMEM((CHUNK,), jnp.int32),
            pltpu.VMEM((L * CE,), jnp.float32),
            pltpu.VMEM((L * C,), jnp.float32),
            pltpu.VMEM((CE,), jnp.float32),
            pltpu.VMEM((C,), jnp.float32),
            pltpu.SemaphoreType.DMA,
        ],
        compiler_params=pltpu.CompilerParams(needs_layout_passes=False),
        interpret=_INTERPRET,
    )
    def seg_kernel(x_hbm, t_hbm, sums_out, cnts_out,
                   xb, tb, stab, ctab, sred, cred, sem):
        wid = lax.axis_index("s") * NC + lax.axis_index("c")
        base = wid * NPX
        zeros = jnp.zeros((L,), jnp.float32)
        ones = jnp.ones((L,), jnp.float32)
        iota = lax.broadcasted_iota(jnp.int32, (L,), 0)

        def zs(i, c):
            stab[pl.ds(i * L, L)] = zeros
            return c

        lax.fori_loop(0, CE, zs, 0)

        def zc(i, c):
            ctab[pl.ds(i * L, L)] = zeros
            return c

        lax.fori_loop(0, C, zc, 0)

        def chunk_body(ci, c):
            off = base + ci * CHUNK
            cps = [
                pltpu.async_copy(x_hbm.at[e, pl.ds(off, CHUNK)], xb.at[e], sem)
                for e in range(E)
            ]
            cpt = pltpu.async_copy(t_hbm.at[pl.ds(off, CHUNK)], tb, sem)
            for cp in cps:
                cp.wait()
            cpt.wait()

            def group(g, c2):
                tv = tb[pl.ds(g * L, L)]
                sbase = iota * CE + tv * E
                plsc.addupdate_scatter(ctab, [iota * C + tv], ones)
                for e in range(E):
                    xv = xb[e, pl.ds(g * L, L)]
                    plsc.addupdate_scatter(stab, [sbase + e], xv)
                return c2

            lax.fori_loop(0, G, group, 0)
            return c

        lax.fori_loop(0, NCHUNK, chunk_body, 0)

        def rs(i, c):
            acc = stab[pl.ds(i * L, L)]
            for lane in range(1, L):
                acc = acc + stab[pl.ds(lane * CE + i * L, L)]
            sred[pl.ds(i * L, L)] = acc
            return c

        lax.fori_loop(0, CE // L, rs, 0)

        def rc(i, c):
            acc = ctab[pl.ds(i * L, L)]
            for lane in range(1, L):
                acc = acc + ctab[pl.ds(lane * C + i * L, L)]
            cred[pl.ds(i * L, L)] = acc
            return c

        lax.fori_loop(0, C // L, rc, 0)

        pltpu.sync_copy(sred, sums_out.at[wid])
        pltpu.sync_copy(cred, cnts_out.at[wid])

    return seg_kernel(x, t)


def _cluster_stats(ps, pc):
    def body(ps_ref, pc_ref, means_ref, aux_ref):
        sums = jnp.sum(ps_ref[...], axis=0)
        counts = jnp.sum(pc_ref[...], axis=0)
        safe = jnp.maximum(counts, 1.0)
        means = sums / safe[:, None]
        means_ref[...] = means
        diff = means[:, None, :] - means[None, :, :]
        d = jnp.sqrt(jnp.sum(diff * diff, axis=-1) + EPS)
        r = lax.broadcasted_iota(jnp.int32, (C, C), 0)
        co = lax.broadcasted_iota(jnp.int32, (C, C), 1)
        d = jnp.where(r == co, d + 2.0 * DELTA_DIST, d)
        hinge = jnp.maximum(2.0 * DELTA_DIST - d, 0.0) ** 2
        dist_term = jnp.sum(hinge) / (C * (C - 1))
        reg = jnp.sum(jnp.sqrt(jnp.sum(means * means, axis=1) + EPS)) / C
        aux_ref[...] = jnp.stack(
            [1.0 / safe, jnp.full((C,), dist_term + GAMMA * reg, jnp.float32)]
        )

    return pl.pallas_call(
        body,
        out_shape=[
            jax.ShapeDtypeStruct((C, ps.shape[2]), jnp.float32),
            jax.ShapeDtypeStruct((2, C), jnp.float32),
        ],
        interpret=_INTERPRET,
    )(ps, pc)


def _var_partials(x, t, means_flat, invc, E, P):
    NPX = P // NW
    NCHUNK = NPX // CHUNK
    G = CHUNK // L

    @functools.partial(
        pl.kernel,
        out_type=jax.ShapeDtypeStruct((NW, L), jnp.float32),
        mesh=_sc_mesh(),
        scratch_types=[
            pltpu.VMEM((E, CHUNK), jnp.float32),
            pltpu.VMEM((CHUNK,), jnp.int32),
            pltpu.VMEM((C * E,), jnp.float32),
            pltpu.VMEM((C,), jnp.float32),
            pltpu.VMEM((L,), jnp.float32),
            pltpu.SemaphoreType.DMA,
        ],
        compiler_params=pltpu.CompilerParams(needs_layout_passes=False),
        interpret=_INTERPRET,
    )
    def var_kernel(x_hbm, t_hbm, m_hbm, ic_hbm, out,
                   xb, tb, mtab, ictab, accb, sem):
        wid = lax.axis_index("s") * NC + lax.axis_index("c")
        base = wid * NPX
        pltpu.sync_copy(m_hbm, mtab)
        pltpu.sync_copy(ic_hbm, ictab)

        def chunk_body(ci, acc):
            off = base + ci * CHUNK
            cps = [
                pltpu.async_copy(x_hbm.at[e, pl.ds(off, CHUNK)], xb.at[e], sem)
                for e in range(E)
            ]
            cpt = pltpu.async_copy(t_hbm.at[pl.ds(off, CHUNK)], tb, sem)
            for cp in cps:
                cp.wait()
            cpt.wait()

            def group(g, acc2):
                tv = tb[pl.ds(g * L, L)]
                midx = tv * E
                d2 = jnp.full((L,), EPS, jnp.float32)
                for e in range(E):
                    xv = xb[e, pl.ds(g * L, L)]
                    mv = plsc.load_gather(mtab, [midx + e])
                    df = xv - mv
                    d2 = d2 + df * df
                ii = plsc.bitcast(d2, jnp.int32)
                ii = jnp.int32(0x5F3759DF) - (ii >> 1)
                y = plsc.bitcast(ii, jnp.float32)
                y = y * (1.5 - 0.5 * d2 * y * y)
                y = y * (1.5 - 0.5 * d2 * y * y)
                y = y * (1.5 - 0.5 * d2 * y * y)
                dist = d2 * y
                h = jnp.maximum(dist - DELTA_VAR, 0.0)
                w = plsc.load_gather(ictab, [tv])
                return acc2 + h * h * w

            return lax.fori_loop(0, G, group, acc)

        acc = lax.fori_loop(0, NCHUNK, chunk_body, jnp.zeros((L,), jnp.float32))
        accb[pl.ds(0, L)] = acc
        pltpu.sync_copy(accb, out.at[wid])

    return var_kernel(x, t, means_flat, invc)


def _finalize(vp, aux):
    def body(vp_ref, aux_ref, out_ref):
        out_ref[...] = (jnp.sum(vp_ref[...]) / C).reshape(1, 1) + aux_ref[1:2, 0:1]

    return pl.pallas_call(
        body,
        out_shape=jax.ShapeDtypeStruct((1, 1), jnp.float32),
        interpret=_INTERPRET,
    )(vp, aux)


def kernel(input_, target):
    B, E, H, W = input_.shape
    P = H * W
    x = input_[B - 1].reshape(E, P)
    t = target[B - 1, 0].reshape(P)
    ps, pc = _seg_stats(x, t, E, P)
    means, aux = _cluster_stats(ps.reshape(NW, C, E), pc)
    vp = _var_partials(x, t, means.reshape(C * E), aux[0], E, P)
    out = _finalize(vp, aux)
    return out[0, 0] * (2.0 / B)


# speedup vs baseline: 6.5970x; 1.5133x over previous
import functools

import jax
import jax.numpy as jnp
from jax import lax
from jax.experimental import pallas as pl
from jax.experimental.pallas import tpu as pltpu
from jax.experimental.pallas import tpu_sc as plsc

DELTA_VAR = 0.5
DELTA_DIST = 2.0
GAMMA = 0.001
EPS = 1e-12
C = 64
L = 16
NC, NS = 2, 16
NW = NC * NS
CHUNK = 1024
MPAD = 33

_INTERPRET = False


def _sc_mesh():
    return plsc.VectorSubcoreMesh(
        core_axis_name="c", subcore_axis_name="s", num_cores=NC, num_subcores=NS
    )


def _seg_stats(x2, t2, E, P, eoff, toff):
    NPX = P // NW
    NCHUNK = NPX // CHUNK
    G = CHUNK // L
    CE = C * E
    SSTR = CE + L + 1
    CSTR = C + 1

    @functools.partial(
        pl.kernel,
        out_type=[
            jax.ShapeDtypeStruct((NW, CE), jnp.float32),
            jax.ShapeDtypeStruct((NW, C), jnp.float32),
        ],
        mesh=_sc_mesh(),
        scratch_types=[
            pltpu.VMEM((2, E, CHUNK), jnp.float32),
            pltpu.VMEM((2, CHUNK), jnp.int32),
            pltpu.VMEM(((L - 1) * SSTR + CE,), jnp.float32),
            pltpu.VMEM(((L - 1) * CSTR + C,), jnp.float32),
            pltpu.VMEM((CE,), jnp.float32),
            pltpu.VMEM((C,), jnp.float32),
            pltpu.SemaphoreType.DMA,
            pltpu.SemaphoreType.DMA,
        ],
        compiler_params=pltpu.CompilerParams(needs_layout_passes=False),
        interpret=_INTERPRET,
    )
    def seg_kernel(x_hbm, t_hbm, sums_out, cnts_out,
                   xb, tb, stab, ctab, sred, cred, sem0, sem1):
        wid = lax.axis_index("s") * NC + lax.axis_index("c")
        base = wid * NPX
        sems = (sem0, sem1)
        zeros = jnp.zeros((L,), jnp.float32)
        ones = jnp.ones((L,), jnp.float32)
        iota = lax.broadcasted_iota(jnp.int32, (L,), 0)

        def zs(i, c):
            stab[pl.ds(i * L, L)] = zeros
            return c

        lax.fori_loop(0, ((L - 1) * SSTR + CE) // L, zs, 0)

        def zc(i, c):
            ctab[pl.ds(i * L, L)] = zeros
            return c

        lax.fori_loop(0, ((L - 1) * CSTR + C) // L, zc, 0)

        def fire(ci, b):
            off = base + ci * CHUNK
            for e in range(E):
                pltpu.async_copy(
                    x_hbm.at[eoff + e, pl.ds(off, CHUNK)], xb.at[b, e], sems[b]
                )
            pltpu.async_copy(
                t_hbm.at[pl.ds(toff + off, CHUNK)], tb.at[b], sems[b]
            )

        def drain(b):
            pltpu.make_async_copy(
                x_hbm.at[pl.ds(eoff, E), pl.ds(base, CHUNK)], xb.at[b], sems[b]
            ).wait()
            pltpu.make_async_copy(
                t_hbm.at[pl.ds(toff, CHUNK)], tb.at[b], sems[b]
            ).wait()

        def compute(b):
            def group(g, c2):
                tv = tb[b, pl.ds(g * L, L)]
                sbase = iota * SSTR + tv * E
                plsc.addupdate_scatter(ctab, [iota * CSTR + tv], ones)
                for e in range(E):
                    xv = xb[b, e, pl.ds(g * L, L)]
                    plsc.addupdate_scatter(stab, [sbase + e], xv)
                return c2

            lax.fori_loop(0, G, group, 0)

        NITER = NCHUNK // 2
        fire(0, 0)

        def pair(j, c):
            fire(2 * j + 1, 1)
            drain(0)
            compute(0)

            @pl.when(j < NITER - 1)
            def _():
                fire(2 * j + 2, 0)

            drain(1)
            compute(1)
            return c

        lax.fori_loop(0, NITER, pair, 0)

        def rs(i, c):
            acc = stab[pl.ds(i * L, L)]
            for lane in range(1, L):
                acc = acc + stab[pl.ds(lane * SSTR + i * L, L)]
            sred[pl.ds(i * L, L)] = acc
            return c

        lax.fori_loop(0, CE // L, rs, 0)

        def rc(i, c):
            acc = ctab[pl.ds(i * L, L)]
            for lane in range(1, L):
                acc = acc + ctab[pl.ds(lane * CSTR + i * L, L)]
            cred[pl.ds(i * L, L)] = acc
            return c

        lax.fori_loop(0, C // L, rc, 0)

        pltpu.sync_copy(sred, sums_out.at[wid])
        pltpu.sync_copy(cred, cnts_out.at[wid])

    return seg_kernel(x2, t2)


def _cluster_stats(ps, pc):
    def body(ps_ref, pc_ref, means_ref, aux_ref):
        sums = jnp.sum(ps_ref[...], axis=0)
        counts = jnp.sum(pc_ref[...], axis=0)
        safe = jnp.maximum(counts, 1.0)
        means = sums / safe[:, None]
        E = sums.shape[1]
        means_ref[...] = jnp.concatenate(
            [means, jnp.zeros((C, MPAD - E), jnp.float32)], axis=1
        )
        diff = means[:, None, :] - means[None, :, :]
        d = jnp.sqrt(jnp.sum(diff * diff, axis=-1) + EPS)
        r = lax.broadcasted_iota(jnp.int32, (C, C), 0)
        co = lax.broadcasted_iota(jnp.int32, (C, C), 1)
        d = jnp.where(r == co, d + 2.0 * DELTA_DIST, d)
        hinge = jnp.maximum(2.0 * DELTA_DIST - d, 0.0) ** 2
        dist_term = jnp.sum(hinge) / (C * (C - 1))
        reg = jnp.sum(jnp.sqrt(jnp.sum(means * means, axis=1) + EPS)) / C
        aux_ref[...] = jnp.stack(
            [1.0 / safe, jnp.full((C,), dist_term + GAMMA * reg, jnp.float32)]
        )

    return pl.pallas_call(
        body,
        out_shape=[
            jax.ShapeDtypeStruct((C, MPAD), jnp.float32),
            jax.ShapeDtypeStruct((2, C), jnp.float32),
        ],
        interpret=_INTERPRET,
    )(ps, pc)


def _var_partials(x2, t2, means_pad, invc, E, P, eoff, toff):
    NPX = P // NW
    NCHUNK = NPX // CHUNK
    G = CHUNK // L

    @functools.partial(
        pl.kernel,
        out_type=jax.ShapeDtypeStruct((NW, L), jnp.float32),
        mesh=_sc_mesh(),
        scratch_types=[
            pltpu.VMEM((2, E, CHUNK), jnp.float32),
            pltpu.VMEM((2, CHUNK), jnp.int32),
            pltpu.VMEM((C, MPAD), jnp.float32),
            pltpu.VMEM((C,), jnp.float32),
            pltpu.VMEM((L,), jnp.float32),
            pltpu.SemaphoreType.DMA,
            pltpu.SemaphoreType.DMA,
        ],
        compiler_params=pltpu.CompilerParams(needs_layout_passes=False),
        interpret=_INTERPRET,
    )
    def var_kernel(x_hbm, t_hbm, m_hbm, ic_hbm, out,
                   xb, tb, mtab, ictab, accb, sem0, sem1):
        wid = lax.axis_index("s") * NC + lax.axis_index("c")
        base = wid * NPX
        sems = (sem0, sem1)
        pltpu.sync_copy(m_hbm, mtab)
        pltpu.sync_copy(ic_hbm, ictab)

        def fire(ci, b):
            off = base + ci * CHUNK
            for e in range(E):
                pltpu.async_copy(
                    x_hbm.at[eoff + e, pl.ds(off, CHUNK)], xb.at[b, e], sems[b]
                )
            pltpu.async_copy(
                t_hbm.at[pl.ds(toff + off, CHUNK)], tb.at[b], sems[b]
            )

        def drain(b):
            pltpu.make_async_copy(
                x_hbm.at[pl.ds(eoff, E), pl.ds(base, CHUNK)], xb.at[b], sems[b]
            ).wait()
            pltpu.make_async_copy(
                t_hbm.at[pl.ds(toff, CHUNK)], tb.at[b], sems[b]
            ).wait()

        def compute(b, acc_in):
            def group(g, acc2):
                tv = tb[b, pl.ds(g * L, L)]
                d2 = jnp.full((L,), EPS, jnp.float32)
                for e in range(E):
                    xv = xb[b, e, pl.ds(g * L, L)]
                    ev = jnp.full((L,), e, jnp.int32)
                    mv = plsc.load_gather(mtab, [tv, ev])
                    df = xv - mv
                    d2 = d2 + df * df
                ii = plsc.bitcast(d2, jnp.int32)
                ii = jnp.int32(0x5F3759DF) - (ii >> 1)
                y = plsc.bitcast(ii, jnp.float32)
                y = y * (1.5 - 0.5 * d2 * y * y)
                y = y * (1.5 - 0.5 * d2 * y * y)
                y = y * (1.5 - 0.5 * d2 * y * y)
                dist = d2 * y
                h = jnp.maximum(dist - DELTA_VAR, 0.0)
                w = plsc.load_gather(ictab, [tv])
                return acc2 + h * h * w

            return lax.fori_loop(0, G, group, acc_in)

        NITER = NCHUNK // 2
        fire(0, 0)

        def pair(j, acc_in):
            fire(2 * j + 1, 1)
            drain(0)
            acc1 = compute(0, acc_in)

            @pl.when(j < NITER - 1)
            def _():
                fire(2 * j + 2, 0)

            drain(1)
            return compute(1, acc1)

        acc = lax.fori_loop(0, NITER, pair, jnp.zeros((L,), jnp.float32))

        accb[pl.ds(0, L)] = acc
        pltpu.sync_copy(accb, out.at[wid])

    return var_kernel(x2, t2, means_pad, invc)


def _finalize(vp, aux):
    def body(vp_ref, aux_ref, out_ref):
        out_ref[...] = (jnp.sum(vp_ref[...]) / C).reshape(1, 1) + aux_ref[1:2, 0:1]

    return pl.pallas_call(
        body,
        out_shape=jax.ShapeDtypeStruct((1, 1), jnp.float32),
        interpret=_INTERPRET,
    )(vp, aux)


def kernel(input_, target):
    B, E, H, W = input_.shape
    P = H * W
    x2 = input_.reshape(B * E, P)
    t2 = target.reshape(B * P)
    eoff = (B - 1) * E
    toff = (B - 1) * P
    ps, pc = _seg_stats(x2, t2, E, P, eoff, toff)
    means_pad, aux = _cluster_stats(ps.reshape(NW, C, E), pc)
    vp = _var_partials(x2, t2, means_pad, aux[0], E, P, eoff, toff)
    out = _finalize(vp, aux)
    return out[0, 0] * (2.0 / B)


# speedup vs baseline: 9.7412x; 1.4766x over previous
import functools

import jax
import jax.numpy as jnp
from jax import lax
from jax.experimental import pallas as pl
from jax.experimental.pallas import tpu as pltpu
from jax.experimental.pallas import tpu_sc as plsc

DELTA_VAR = 0.5
DELTA_DIST = 2.0
GAMMA = 0.001
EPS = 1e-12
C = 64
L = 16
NC, NS = 2, 16
NW = NC * NS
CHUNK = 1024

_INTERPRET = False


def _sc_mesh():
    return plsc.VectorSubcoreMesh(
        core_axis_name="c", subcore_axis_name="s", num_cores=NC, num_subcores=NS
    )


def _seg_stats(x, t, E, P):
    NPX = P // NW
    NCHUNK = NPX // CHUNK
    G = CHUNK // L
    CE = C * E
    SSTR = CE + L + 1
    CSTR = C + 1

    @functools.partial(
        pl.kernel,
        out_type=[
            jax.ShapeDtypeStruct((NW, CE), jnp.float32),
            jax.ShapeDtypeStruct((NW, C), jnp.float32),
        ],
        mesh=_sc_mesh(),
        scratch_types=[
            pltpu.VMEM((2 * E * CHUNK,), jnp.float32),
            pltpu.VMEM((2 * CHUNK,), jnp.int32),
            pltpu.VMEM(((L - 1) * SSTR + CE,), jnp.float32),
            pltpu.VMEM(((L - 1) * CSTR + C,), jnp.float32),
            pltpu.VMEM((CE,), jnp.float32),
            pltpu.VMEM((C,), jnp.float32),
            pltpu.SemaphoreType.DMA,
            pltpu.SemaphoreType.DMA,
        ],
        compiler_params=pltpu.CompilerParams(needs_layout_passes=False),
        interpret=_INTERPRET,
    )
    def seg_kernel(x_hbm, t_hbm, sums_out, cnts_out,
                   xb, tb, stab, ctab, sred, cred, sem0, sem1):
        wid = lax.axis_index("s") * NC + lax.axis_index("c")
        base = wid * NPX
        sems = (sem0, sem1)
        zeros = jnp.zeros((L,), jnp.float32)
        ones = jnp.ones((L,), jnp.float32)
        iota = lax.broadcasted_iota(jnp.int32, (L,), 0)

        def zs(i, c):
            stab[pl.ds(i * L, L)] = zeros
            return c

        lax.fori_loop(0, ((L - 1) * SSTR + CE) // L, zs, 0)

        def zc(i, c):
            ctab[pl.ds(i * L, L)] = zeros
            return c

        lax.fori_loop(0, ((L - 1) * CSTR + C) // L, zc, 0)

        def fire(ci, b):
            off = base + ci * CHUNK
            for e in range(E):
                pltpu.async_copy(
                    x_hbm.at[e, pl.ds(off, CHUNK)],
                    xb.at[pl.ds((b * E + e) * CHUNK, CHUNK)],
                    sems[b],
                )
            pltpu.async_copy(
                t_hbm.at[pl.ds(off, CHUNK)],
                tb.at[pl.ds(b * CHUNK, CHUNK)],
                sems[b],
            )

        def drain(b):
            pltpu.make_async_copy(
                x_hbm.at[0, pl.ds(0, E * CHUNK)],
                xb.at[pl.ds(b * E * CHUNK, E * CHUNK)],
                sems[b],
            ).wait()
            pltpu.make_async_copy(
                t_hbm.at[pl.ds(base, CHUNK)],
                tb.at[pl.ds(b * CHUNK, CHUNK)],
                sems[b],
            ).wait()

        def compute(b):
            def group(gi, c2):
                for u in range(2):
                    g = gi * 2 + u
                    tv = tb[pl.ds(b * CHUNK + g * L, L)]
                    sbase = iota * SSTR + tv * E
                    plsc.addupdate_scatter(ctab, [iota * CSTR + tv], ones)
                    for e in range(E):
                        xv = xb[pl.ds((b * E + e) * CHUNK + g * L, L)]
                        plsc.addupdate_scatter(stab, [sbase + e], xv)
                return c2

            lax.fori_loop(0, G // 2, group, 0)

        NITER = NCHUNK // 2
        fire(0, 0)

        def pair(j, c):
            fire(2 * j + 1, 1)
            drain(0)
            compute(0)

            @pl.when(j < NITER - 1)
            def _():
                fire(2 * j + 2, 0)

            drain(1)
            compute(1)
            return c

        lax.fori_loop(0, NITER, pair, 0)

        def rs(i, c):
            acc = stab[pl.ds(i * L, L)]
            for lane in range(1, L):
                acc = acc + stab[pl.ds(lane * SSTR + i * L, L)]
            sred[pl.ds(i * L, L)] = acc
            return c

        lax.fori_loop(0, CE // L, rs, 0)

        def rc(i, c):
            acc = ctab[pl.ds(i * L, L)]
            for lane in range(1, L):
                acc = acc + ctab[pl.ds(lane * CSTR + i * L, L)]
            cred[pl.ds(i * L, L)] = acc
            return c

        lax.fori_loop(0, C // L, rc, 0)

        pltpu.sync_copy(sred, sums_out.at[wid])
        pltpu.sync_copy(cred, cnts_out.at[wid])

    return seg_kernel(x, t)


def _cluster_stats(ps, pc):
    def body(ps_ref, pc_ref, mrep_ref, aux_ref):
        sums = jnp.sum(ps_ref[...], axis=0)
        counts = jnp.sum(pc_ref[...], axis=0)
        safe = jnp.maximum(counts, 1.0)
        invc = 1.0 / safe
        means = sums / safe[:, None]
        E = sums.shape[1]
        aug = jnp.concatenate([means, invc[:, None]], axis=1)
        mrep_ref[...] = jnp.broadcast_to(aug[:, :, None], (C, E + 1, L))
        diff = means[:, None, :] - means[None, :, :]
        d = jnp.sqrt(jnp.sum(diff * diff, axis=-1) + EPS)
        r = lax.broadcasted_iota(jnp.int32, (C, C), 0)
        co = lax.broadcasted_iota(jnp.int32, (C, C), 1)
        d = jnp.where(r == co, d + 2.0 * DELTA_DIST, d)
        hinge = jnp.maximum(2.0 * DELTA_DIST - d, 0.0) ** 2
        dist_term = jnp.sum(hinge) / (C * (C - 1))
        reg = jnp.sum(jnp.sqrt(jnp.sum(means * means, axis=1) + EPS)) / C
        aux_ref[...] = jnp.stack(
            [invc, jnp.full((C,), dist_term + GAMMA * reg, jnp.float32)]
        )

    E = ps.shape[2]
    return pl.pallas_call(
        body,
        out_shape=[
            jax.ShapeDtypeStruct((C, E + 1, L), jnp.float32),
            jax.ShapeDtypeStruct((2, C), jnp.float32),
        ],
        interpret=_INTERPRET,
    )(ps, pc)


def _var_partials(x, t, mrep, E, P):
    NPX = P // NW
    NCHUNK = NPX // CHUNK
    G = CHUNK // L
    CE = C * E

    @functools.partial(
        pl.kernel,
        out_type=jax.ShapeDtypeStruct((NW, L), jnp.float32),
        mesh=_sc_mesh(),
        scratch_types=[
            pltpu.VMEM((2 * E * CHUNK,), jnp.float32),
            pltpu.VMEM((2 * CHUNK,), jnp.int32),
            pltpu.VMEM((C * (E + 1) * L,), jnp.float32),
            pltpu.VMEM((L,), jnp.float32),
            pltpu.SemaphoreType.DMA,
            pltpu.SemaphoreType.DMA,
        ],
        compiler_params=pltpu.CompilerParams(needs_layout_passes=False),
        interpret=_INTERPRET,
    )
    def var_kernel(x_hbm, t_hbm, m_hbm, out, xb, tb, mtab, accb, sem0, sem1):
        wid = lax.axis_index("s") * NC + lax.axis_index("c")
        base = wid * NPX
        sems = (sem0, sem1)
        iota = lax.broadcasted_iota(jnp.int32, (L,), 0)
        pltpu.sync_copy(m_hbm, mtab)

        def fire(ci, b):
            off = base + ci * CHUNK
            for e in range(E):
                pltpu.async_copy(
                    x_hbm.at[e, pl.ds(off, CHUNK)],
                    xb.at[pl.ds((b * E + e) * CHUNK, CHUNK)],
                    sems[b],
                )
            pltpu.async_copy(
                t_hbm.at[pl.ds(off, CHUNK)],
                tb.at[pl.ds(b * CHUNK, CHUNK)],
                sems[b],
            )

        def drain(b):
            pltpu.make_async_copy(
                x_hbm.at[0, pl.ds(0, E * CHUNK)],
                xb.at[pl.ds(b * E * CHUNK, E * CHUNK)],
                sems[b],
            ).wait()
            pltpu.make_async_copy(
                t_hbm.at[pl.ds(base, CHUNK)],
                tb.at[pl.ds(b * CHUNK, CHUNK)],
                sems[b],
            ).wait()

        def compute(b, acc_in):
            def group(gi, acc2):
                for u in range(2):
                    g = gi * 2 + u
                    tv = tb[pl.ds(b * CHUNK + g * L, L)]
                    gb = tv * ((E + 1) * L) + iota
                    d2 = jnp.full((L,), EPS, jnp.float32)
                    for e in range(E):
                        xv = xb[pl.ds((b * E + e) * CHUNK + g * L, L)]
                        mv = plsc.load_gather(mtab, [gb + e * L])
                        df = xv - mv
                        d2 = d2 + df * df
                    ii = plsc.bitcast(d2, jnp.int32)
                    ii = jnp.int32(0x5F3759DF) - (ii >> 1)
                    y = plsc.bitcast(ii, jnp.float32)
                    y = y * (1.5 - 0.5 * d2 * y * y)
                    y = y * (1.5 - 0.5 * d2 * y * y)
                    y = y * (1.5 - 0.5 * d2 * y * y)
                    dist = d2 * y
                    h = jnp.maximum(dist - DELTA_VAR, 0.0)
                    w = plsc.load_gather(mtab, [gb + E * L])
                    acc2 = acc2 + h * h * w
                return acc2

            return lax.fori_loop(0, G // 2, group, acc_in)

        NITER = NCHUNK // 2
        fire(0, 0)

        def pair(j, acc_in):
            fire(2 * j + 1, 1)
            drain(0)
            acc1 = compute(0, acc_in)

            @pl.when(j < NITER - 1)
            def _():
                fire(2 * j + 2, 0)

            drain(1)
            return compute(1, acc1)

        acc = lax.fori_loop(0, NITER, pair, jnp.zeros((L,), jnp.float32))

        accb[pl.ds(0, L)] = acc
        pltpu.sync_copy(accb, out.at[wid])

    return var_kernel(x, t, mrep)


def _finalize(vp, aux):
    def body(vp_ref, aux_ref, out_ref):
        out_ref[...] = (jnp.sum(vp_ref[...]) / C).reshape(1, 1) + aux_ref[1:2, 0:1]

    return pl.pallas_call(
        body,
        out_shape=jax.ShapeDtypeStruct((1, 1), jnp.float32),
        interpret=_INTERPRET,
    )(vp, aux)


def kernel(input_, target):
    B, E, H, W = input_.shape
    P = H * W
    x = input_[B - 1].reshape(E, P)
    t = target[B - 1, 0].reshape(P)
    ps, pc = _seg_stats(x, t, E, P)
    mrep, aux = _cluster_stats(ps.reshape(NW, C, E), pc)
    vp = _var_partials(x, t, mrep.reshape(-1), E, P)
    out = _finalize(vp, aux)
    return out[0, 0] * (2.0 / B)


# speedup vs baseline: 14.1505x; 1.4526x over previous
import functools

import jax
import jax.numpy as jnp
from jax import lax
from jax.experimental import pallas as pl
from jax.experimental.pallas import tpu as pltpu
from jax.experimental.pallas import tpu_sc as plsc

DELTA_VAR = 0.5
DELTA_DIST = 2.0
GAMMA = 0.001
EPS = 1e-12
C = 64
L = 16
NC, NS = 2, 16
NW = NC * NS
TR, TW = 8, 128
CHUNK = TR * TW

_INTERPRET = False


def _sc_mesh():
    return plsc.VectorSubcoreMesh(
        core_axis_name="c", subcore_axis_name="s", num_cores=NC, num_subcores=NS
    )


def _chunk_origin(wid, ci, H, W):
    rows_per_sub = H // NW
    tiles_c = W // TW
    r0 = wid * rows_per_sub + (ci // tiles_c) * TR
    c0 = (ci % tiles_c) * TW
    return r0, c0


def _seg_stats(inp, tgt):
    B, E, H, W = inp.shape
    NPX = (H // NW) * W
    NCHUNK = NPX // CHUNK
    G = CHUNK // L
    CE = C * E
    SSTR = CE + L + 1
    CSTR = C + 1

    @functools.partial(
        pl.kernel,
        out_type=[
            jax.ShapeDtypeStruct((NW, CE), jnp.float32),
            jax.ShapeDtypeStruct((NW, C), jnp.float32),
        ],
        mesh=_sc_mesh(),
        scratch_types=[
            pltpu.VMEM((2 * E * TR, TW), jnp.float32),
            pltpu.VMEM((2 * TR, TW), jnp.int32),
            pltpu.VMEM(((L - 1) * SSTR + CE,), jnp.float32),
            pltpu.VMEM(((L - 1) * CSTR + C,), jnp.float32),
            pltpu.VMEM((CE,), jnp.float32),
            pltpu.VMEM((C,), jnp.float32),
            pltpu.SemaphoreType.DMA,
            pltpu.SemaphoreType.DMA,
        ],
        compiler_params=pltpu.CompilerParams(needs_layout_passes=False),
        interpret=_INTERPRET,
    )
    def seg_kernel(x_hbm, t_hbm, sums_out, cnts_out,
                   xb, tb, stab, ctab, sred, cred, sem0, sem1):
        wid = lax.axis_index("s") * NC + lax.axis_index("c")
        sems = (sem0, sem1)
        zeros = jnp.zeros((L,), jnp.float32)
        ones = jnp.ones((L,), jnp.float32)
        iota = lax.broadcasted_iota(jnp.int32, (L,), 0)

        def zs(i, c):
            stab[pl.ds(i * L, L)] = zeros
            return c

        lax.fori_loop(0, ((L - 1) * SSTR + CE) // L, zs, 0)

        def zc(i, c):
            ctab[pl.ds(i * L, L)] = zeros
            return c

        lax.fori_loop(0, ((L - 1) * CSTR + C) // L, zc, 0)

        def fire(ci, b):
            r0, c0 = _chunk_origin(wid, ci, H, W)
            for e in range(E):
                pltpu.async_copy(
                    x_hbm.at[B - 1, e, pl.ds(r0, TR), pl.ds(c0, TW)],
                    xb.at[pl.ds((b * E + e) * TR, TR), :],
                    sems[b],
                )
            pltpu.async_copy(
                t_hbm.at[B - 1, 0, pl.ds(r0, TR), pl.ds(c0, TW)],
                tb.at[pl.ds(b * TR, TR), :],
                sems[b],
            )

        def drain(b):
            pltpu.make_async_copy(
                x_hbm.at[0, 0, pl.ds(0, E * TR), pl.ds(0, TW)],
                xb.at[pl.ds(b * E * TR, E * TR), :],
                sems[b],
            ).wait()
            pltpu.make_async_copy(
                t_hbm.at[0, 0, pl.ds(0, TR), pl.ds(0, TW)],
                tb.at[pl.ds(b * TR, TR), :],
                sems[b],
            ).wait()

        def compute(b):
            def group(g, c2):
                row = g // (TW // L)
                cc = (g % (TW // L)) * L
                tv = tb[b * TR + row, pl.ds(cc, L)]
                sbase = iota * SSTR + tv * E
                plsc.addupdate_scatter(ctab, [iota * CSTR + tv], ones)
                for e in range(E):
                    xv = xb[(b * E + e) * TR + row, pl.ds(cc, L)]
                    plsc.addupdate_scatter(stab, [sbase + e], xv)
                return c2

            lax.fori_loop(0, G, group, 0)

        NITER = NCHUNK // 2
        fire(0, 0)

        def pair(j, c):
            fire(2 * j + 1, 1)
            drain(0)
            compute(0)

            @pl.when(j < NITER - 1)
            def _():
                fire(2 * j + 2, 0)

            drain(1)
            compute(1)
            return c

        lax.fori_loop(0, NITER, pair, 0)

        def rs(i, c):
            acc = stab[pl.ds(i * L, L)]
            for lane in range(1, L):
                acc = acc + stab[pl.ds(lane * SSTR + i * L, L)]
            sred[pl.ds(i * L, L)] = acc
            return c

        lax.fori_loop(0, CE // L, rs, 0)

        def rc(i, c):
            acc = ctab[pl.ds(i * L, L)]
            for lane in range(1, L):
                acc = acc + ctab[pl.ds(lane * CSTR + i * L, L)]
            cred[pl.ds(i * L, L)] = acc
            return c

        lax.fori_loop(0, C // L, rc, 0)

        pltpu.sync_copy(sred, sums_out.at[wid])
        pltpu.sync_copy(cred, cnts_out.at[wid])

    return seg_kernel(inp, tgt)


def _cluster_stats(ps, pc):
    def body(ps_ref, pc_ref, mrep_ref, aux_ref):
        sums = jnp.sum(ps_ref[...], axis=0)
        counts = jnp.sum(pc_ref[...], axis=0)
        safe = jnp.maximum(counts, 1.0)
        invc = 1.0 / safe
        means = sums / safe[:, None]
        E = sums.shape[1]
        aug = jnp.concatenate([means, invc[:, None]], axis=1)
        mrep_ref[...] = jnp.broadcast_to(aug[:, :, None], (C, E + 1, L))
        diff = means[:, None, :] - means[None, :, :]
        d = jnp.sqrt(jnp.sum(diff * diff, axis=-1) + EPS)
        r = lax.broadcasted_iota(jnp.int32, (C, C), 0)
        co = lax.broadcasted_iota(jnp.int32, (C, C), 1)
        d = jnp.where(r == co, d + 2.0 * DELTA_DIST, d)
        hinge = jnp.maximum(2.0 * DELTA_DIST - d, 0.0) ** 2
        dist_term = jnp.sum(hinge) / (C * (C - 1))
        reg = jnp.sum(jnp.sqrt(jnp.sum(means * means, axis=1) + EPS)) / C
        aux_ref[...] = jnp.stack(
            [invc, jnp.full((C,), dist_term + GAMMA * reg, jnp.float32)]
        )

    E = ps.shape[2]
    return pl.pallas_call(
        body,
        out_shape=[
            jax.ShapeDtypeStruct((C, E + 1, L), jnp.float32),
            jax.ShapeDtypeStruct((2, C), jnp.float32),
        ],
        interpret=_INTERPRET,
    )(ps, pc)


def _var_partials(inp, tgt, mrep):
    B, E, H, W = inp.shape
    NPX = (H // NW) * W
    NCHUNK = NPX // CHUNK
    G = CHUNK // L
    CE = C * E

    @functools.partial(
        pl.kernel,
        out_type=jax.ShapeDtypeStruct((NW, L), jnp.float32),
        mesh=_sc_mesh(),
        scratch_types=[
            pltpu.VMEM((2 * E * TR, TW), jnp.float32),
            pltpu.VMEM((2 * TR, TW), jnp.int32),
            pltpu.VMEM((C * (E + 1) * L,), jnp.float32),
            pltpu.VMEM((L,), jnp.float32),
            pltpu.SemaphoreType.DMA,
            pltpu.SemaphoreType.DMA,
        ],
        compiler_params=pltpu.CompilerParams(needs_layout_passes=False),
        interpret=_INTERPRET,
    )
    def var_kernel(x_hbm, t_hbm, m_hbm, out, xb, tb, mtab, accb, sem0, sem1):
        wid = lax.axis_index("s") * NC + lax.axis_index("c")
        sems = (sem0, sem1)
        iota = lax.broadcasted_iota(jnp.int32, (L,), 0)
        pltpu.sync_copy(m_hbm, mtab)

        def fire(ci, b):
            r0, c0 = _chunk_origin(wid, ci, H, W)
            for e in range(E):
                pltpu.async_copy(
                    x_hbm.at[B - 1, e, pl.ds(r0, TR), pl.ds(c0, TW)],
                    xb.at[pl.ds((b * E + e) * TR, TR), :],
                    sems[b],
                )
            pltpu.async_copy(
                t_hbm.at[B - 1, 0, pl.ds(r0, TR), pl.ds(c0, TW)],
                tb.at[pl.ds(b * TR, TR), :],
                sems[b],
            )

        def drain(b):
            pltpu.make_async_copy(
                x_hbm.at[0, 0, pl.ds(0, E * TR), pl.ds(0, TW)],
                xb.at[pl.ds(b * E * TR, E * TR), :],
                sems[b],
            ).wait()
            pltpu.make_async_copy(
                t_hbm.at[0, 0, pl.ds(0, TR), pl.ds(0, TW)],
                tb.at[pl.ds(b * TR, TR), :],
                sems[b],
            ).wait()

        def compute(b, acc_in):
            def group(g, acc2):
                row = g // (TW // L)
                cc = (g % (TW // L)) * L
                tv = tb[b * TR + row, pl.ds(cc, L)]
                gb = tv * ((E + 1) * L) + iota
                d2 = jnp.full((L,), EPS, jnp.float32)
                for e in range(E):
                    xv = xb[(b * E + e) * TR + row, pl.ds(cc, L)]
                    mv = plsc.load_gather(mtab, [gb + e * L])
                    df = xv - mv
                    d2 = d2 + df * df
                ii = plsc.bitcast(d2, jnp.int32)
                ii = jnp.int32(0x5F3759DF) - (ii >> 1)
                y = plsc.bitcast(ii, jnp.float32)
                y = y * (1.5 - 0.5 * d2 * y * y)
                y = y * (1.5 - 0.5 * d2 * y * y)
                y = y * (1.5 - 0.5 * d2 * y * y)
                dist = d2 * y
                h = jnp.maximum(dist - DELTA_VAR, 0.0)
                w = plsc.load_gather(mtab, [gb + E * L])
                return acc2 + h * h * w

            return lax.fori_loop(0, G, group, acc_in)

        NITER = NCHUNK // 2
        fire(0, 0)

        def pair(j, acc_in):
            fire(2 * j + 1, 1)
            drain(0)
            acc1 = compute(0, acc_in)

            @pl.when(j < NITER - 1)
            def _():
                fire(2 * j + 2, 0)

            drain(1)
            return compute(1, acc1)

        acc = lax.fori_loop(0, NITER, pair, jnp.zeros((L,), jnp.float32))

        accb[pl.ds(0, L)] = acc
        pltpu.sync_copy(accb, out.at[wid])

    return var_kernel(inp, tgt, mrep)


def _finalize(vp, aux):
    def body(vp_ref, aux_ref, out_ref):
        out_ref[...] = (jnp.sum(vp_ref[...]) / C).reshape(1, 1) + aux_ref[1:2, 0:1]

    return pl.pallas_call(
        body,
        out_shape=jax.ShapeDtypeStruct((1, 1), jnp.float32),
        interpret=_INTERPRET,
    )(vp, aux)


def kernel(input_, target):
    B = input_.shape[0]
    E = input_.shape[1]
    ps, pc = _seg_stats(input_, target)
    mrep, aux = _cluster_stats(ps.reshape(NW, C, E), pc)
    vp = _var_partials(input_, target, mrep.reshape(-1))
    out = _finalize(vp, aux)
    return out[0, 0] * (2.0 / B)


# speedup vs baseline: 14.2505x; 1.0071x over previous
import functools

import jax
import jax.numpy as jnp
from jax import lax
from jax.experimental import pallas as pl
from jax.experimental.pallas import tpu as pltpu
from jax.experimental.pallas import tpu_sc as plsc

DELTA_VAR = 0.5
DELTA_DIST = 2.0
GAMMA = 0.001
EPS = 1e-12
C = 64
L = 16
NC, NS = 2, 16
NW = NC * NS
TR, TW = 8, 128
CHUNK = TR * TW

_INTERPRET = False


def _sc_mesh():
    return plsc.VectorSubcoreMesh(
        core_axis_name="c", subcore_axis_name="s", num_cores=NC, num_subcores=NS
    )


def _chunk_origin(wid, ci, H, W):
    rows_per_sub = H // NW
    tiles_c = W // TW
    r0 = wid * rows_per_sub + (ci // tiles_c) * TR
    c0 = (ci % tiles_c) * TW
    return r0, c0


def _seg_stats(inp, tgt):
    B, E, H, W = inp.shape
    NPX = (H // NW) * W
    NCHUNK = NPX // CHUNK
    G = CHUNK // L
    CE = C * E
    EK = 16
    NK = E // EK
    SSTR = C * EK + 1
    SSZ = (L - 1) * SSTR + C * EK + 1
    CSTR = C + 1

    @functools.partial(
        pl.kernel,
        out_type=[
            jax.ShapeDtypeStruct((NW, CE), jnp.float32),
            jax.ShapeDtypeStruct((NW, C), jnp.float32),
        ],
        mesh=_sc_mesh(),
        scratch_types=[
            pltpu.VMEM((2 * E * TR, TW), jnp.float32),
            pltpu.VMEM((2 * TR, TW), jnp.int32),
        ]
        + [pltpu.VMEM((SSZ,), jnp.float32) for _ in range(NK)]
        + [
            pltpu.VMEM(((L - 1) * CSTR + C,), jnp.float32),
            pltpu.VMEM((CE,), jnp.float32),
            pltpu.VMEM((C,), jnp.float32),
            pltpu.SemaphoreType.DMA,
            pltpu.SemaphoreType.DMA,
        ],
        compiler_params=pltpu.CompilerParams(needs_layout_passes=False),
        interpret=_INTERPRET,
    )
    def seg_kernel(x_hbm, t_hbm, sums_out, cnts_out,
                   xb, tb, *rest):
        stabs = rest[:NK]
        ctab, sred, cred, sem0, sem1 = rest[NK:]
        wid = lax.axis_index("s") * NC + lax.axis_index("c")
        sems = (sem0, sem1)
        zeros = jnp.zeros((L,), jnp.float32)
        ones = jnp.ones((L,), jnp.float32)
        iota = lax.broadcasted_iota(jnp.int32, (L,), 0)

        def zs(i, c):
            for k in range(NK):
                stabs[k][pl.ds(i * L, L)] = zeros
            return c

        lax.fori_loop(0, SSZ // L, zs, 0)

        def zc(i, c):
            ctab[pl.ds(i * L, L)] = zeros
            return c

        lax.fori_loop(0, ((L - 1) * CSTR + C) // L, zc, 0)

        def fire(ci, b):
            r0, c0 = _chunk_origin(wid, ci, H, W)
            for e in range(E):
                pltpu.async_copy(
                    x_hbm.at[B - 1, e, pl.ds(r0, TR), pl.ds(c0, TW)],
                    xb.at[pl.ds((b * E + e) * TR, TR), :],
                    sems[b],
                )
            pltpu.async_copy(
                t_hbm.at[B - 1, 0, pl.ds(r0, TR), pl.ds(c0, TW)],
                tb.at[pl.ds(b * TR, TR), :],
                sems[b],
            )

        def drain(b):
            pltpu.make_async_copy(
                x_hbm.at[0, 0, pl.ds(0, E * TR), pl.ds(0, TW)],
                xb.at[pl.ds(b * E * TR, E * TR), :],
                sems[b],
            ).wait()
            pltpu.make_async_copy(
                t_hbm.at[0, 0, pl.ds(0, TR), pl.ds(0, TW)],
                tb.at[pl.ds(b * TR, TR), :],
                sems[b],
            ).wait()

        def compute(b):
            def group(g, c2):
                row = g // (TW // L)
                cc = (g % (TW // L)) * L
                tv = tb[b * TR + row, pl.ds(cc, L)]
                sbase = iota * SSTR + tv * EK
                plsc.addupdate_scatter(ctab, [iota * CSTR + tv], ones)
                for e in range(E):
                    xv = xb[(b * E + e) * TR + row, pl.ds(cc, L)]
                    plsc.addupdate_scatter(
                        stabs[e // EK], [sbase + (e % EK)], xv
                    )
                return c2

            lax.fori_loop(0, G, group, 0)

        NITER = NCHUNK // 2
        fire(0, 0)

        def pair(j, c):
            fire(2 * j + 1, 1)
            drain(0)
            compute(0)

            @pl.when(j < NITER - 1)
            def _():
                fire(2 * j + 2, 0)

            drain(1)
            compute(1)
            return c

        lax.fori_loop(0, NITER, pair, 0)

        def rs(i, c):
            for k in range(NK):
                acc = stabs[k][pl.ds(i * L, L)]
                for lane in range(1, L):
                    acc = acc + stabs[k][pl.ds(lane * SSTR + i * L, L)]
                sred[pl.ds(k * (C * EK) + i * L, L)] = acc
            return c

        lax.fori_loop(0, C * EK // L, rs, 0)

        def rc(i, c):
            acc = ctab[pl.ds(i * L, L)]
            for lane in range(1, L):
                acc = acc + ctab[pl.ds(lane * CSTR + i * L, L)]
            cred[pl.ds(i * L, L)] = acc
            return c

        lax.fori_loop(0, C // L, rc, 0)

        pltpu.sync_copy(sred, sums_out.at[wid])
        pltpu.sync_copy(cred, cnts_out.at[wid])

    return seg_kernel(inp, tgt)


def _cluster_stats(ps, pc):
    def body(ps_ref, pc_ref, mrep_ref, aux_ref):
        s4 = jnp.sum(ps_ref[...], axis=0)
        sums = jnp.concatenate([s4[k] for k in range(s4.shape[0])], axis=1)
        counts = jnp.sum(pc_ref[...], axis=0)
        safe = jnp.maximum(counts, 1.0)
        invc = 1.0 / safe
        means = sums / safe[:, None]
        E = sums.shape[1]
        aug = jnp.concatenate([means, invc[:, None]], axis=1)
        mrep_ref[...] = jnp.broadcast_to(aug[:, :, None], (C, E + 1, L))
        diff = means[:, None, :] - means[None, :, :]
        d = jnp.sqrt(jnp.sum(diff * diff, axis=-1) + EPS)
        r = lax.broadcasted_iota(jnp.int32, (C, C), 0)
        co = lax.broadcasted_iota(jnp.int32, (C, C), 1)
        d = jnp.where(r == co, d + 2.0 * DELTA_DIST, d)
        hinge = jnp.maximum(2.0 * DELTA_DIST - d, 0.0) ** 2
        dist_term = jnp.sum(hinge) / (C * (C - 1))
        reg = jnp.sum(jnp.sqrt(jnp.sum(means * means, axis=1) + EPS)) / C
        aux_ref[...] = jnp.stack(
            [invc, jnp.full((C,), dist_term + GAMMA * reg, jnp.float32)]
        )

    E = ps.shape[1] * ps.shape[3]
    return pl.pallas_call(
        body,
        out_shape=[
            jax.ShapeDtypeStruct((C, E + 1, L), jnp.float32),
            jax.ShapeDtypeStruct((2, C), jnp.float32),
        ],
        interpret=_INTERPRET,
    )(ps, pc)


def _var_partials(inp, tgt, mrep):
    B, E, H, W = inp.shape
    NPX = (H // NW) * W
    NCHUNK = NPX // CHUNK
    G = CHUNK // L
    CE = C * E

    @functools.partial(
        pl.kernel,
        out_type=jax.ShapeDtypeStruct((NW, L), jnp.float32),
        mesh=_sc_mesh(),
        scratch_types=[
            pltpu.VMEM((2 * E * TR, TW), jnp.float32),
            pltpu.VMEM((2 * TR, TW), jnp.int32),
            pltpu.VMEM((C * (E + 1) * L,), jnp.float32),
            pltpu.VMEM((L,), jnp.float32),
            pltpu.SemaphoreType.DMA,
            pltpu.SemaphoreType.DMA,
        ],
        compiler_params=pltpu.CompilerParams(needs_layout_passes=False),
        interpret=_INTERPRET,
    )
    def var_kernel(x_hbm, t_hbm, m_hbm, out, xb, tb, mtab, accb, sem0, sem1):
        wid = lax.axis_index("s") * NC + lax.axis_index("c")
        sems = (sem0, sem1)
        iota = lax.broadcasted_iota(jnp.int32, (L,), 0)
        pltpu.sync_copy(m_hbm, mtab)

        def fire(ci, b):
            r0, c0 = _chunk_origin(wid, ci, H, W)
            for e in range(E):
                pltpu.async_copy(
                    x_hbm.at[B - 1, e, pl.ds(r0, TR), pl.ds(c0, TW)],
                    xb.at[pl.ds((b * E + e) * TR, TR), :],
                    sems[b],
                )
            pltpu.async_copy(
                t_hbm.at[B - 1, 0, pl.ds(r0, TR), pl.ds(c0, TW)],
                tb.at[pl.ds(b * TR, TR), :],
                sems[b],
            )

        def drain(b):
            pltpu.make_async_copy(
                x_hbm.at[0, 0, pl.ds(0, E * TR), pl.ds(0, TW)],
                xb.at[pl.ds(b * E * TR, E * TR), :],
                sems[b],
            ).wait()
            pltpu.make_async_copy(
                t_hbm.at[0, 0, pl.ds(0, TR), pl.ds(0, TW)],
                tb.at[pl.ds(b * TR, TR), :],
                sems[b],
            ).wait()

        def compute(b, acc_in):
            def group(g, acc2):
                row = g // (TW // L)
                cc = (g % (TW // L)) * L
                tv = tb[b * TR + row, pl.ds(cc, L)]
                gb = tv * ((E + 1) * L) + iota
                d2 = jnp.full((L,), EPS, jnp.float32)
                for e in range(E):
                    xv = xb[(b * E + e) * TR + row, pl.ds(cc, L)]
                    mv = plsc.load_gather(mtab, [gb + e * L])
                    df = xv - mv
                    d2 = d2 + df * df
                ii = plsc.bitcast(d2, jnp.int32)
                ii = jnp.int32(0x5F3759DF) - (ii >> 1)
                y = plsc.bitcast(ii, jnp.float32)
                y = y * (1.5 - 0.5 * d2 * y * y)
                y = y * (1.5 - 0.5 * d2 * y * y)
                y = y * (1.5 - 0.5 * d2 * y * y)
                dist = d2 * y
                h = jnp.maximum(dist - DELTA_VAR, 0.0)
                w = plsc.load_gather(mtab, [gb + E * L])
                return acc2 + h * h * w

            return lax.fori_loop(0, G, group, acc_in)

        NITER = NCHUNK // 2
        fire(0, 0)

        def pair(j, acc_in):
            fire(2 * j + 1, 1)
            drain(0)
            acc1 = compute(0, acc_in)

            @pl.when(j < NITER - 1)
            def _():
                fire(2 * j + 2, 0)

            drain(1)
            return compute(1, acc1)

        acc = lax.fori_loop(0, NITER, pair, jnp.zeros((L,), jnp.float32))

        accb[pl.ds(0, L)] = acc
        pltpu.sync_copy(accb, out.at[wid])

    return var_kernel(inp, tgt, mrep)


def _finalize(vp, aux):
    def body(vp_ref, aux_ref, out_ref):
        out_ref[...] = (jnp.sum(vp_ref[...]) / C).reshape(1, 1) + aux_ref[1:2, 0:1]

    return pl.pallas_call(
        body,
        out_shape=jax.ShapeDtypeStruct((1, 1), jnp.float32),
        interpret=_INTERPRET,
    )(vp, aux)


def kernel(input_, target):
    B = input_.shape[0]
    E = input_.shape[1]
    ps, pc = _seg_stats(input_, target)
    mrep, aux = _cluster_stats(ps.reshape(NW, E // 16, C, 16), pc)
    vp = _var_partials(input_, target, mrep.reshape(-1))
    out = _finalize(vp, aux)
    return out[0, 0] * (2.0 / B)
